# baseline (speedup 1.0000x reference)
import functools

import jax
import jax.numpy as jnp
from jax import lax
from jax.experimental import pallas as pl
from jax.experimental.pallas import tpu as pltpu
from jax.experimental.pallas import tpu_sc as plsc

_BATCH = 16384
_WIDTH = 64


def _build_gather():
    info = plsc.get_sparse_core_info()
    nc, ns, nl = info.num_cores, info.num_subcores, info.num_lanes
    nw = nc * ns
    bpw = _BATCH // nw
    nch = 4
    chunk = bpw // nch

    mesh = plsc.VectorSubcoreMesh(core_axis_name="c", subcore_axis_name="s")

    @functools.partial(
        pl.kernel,
        mesh=mesh,
        out_type=jax.ShapeDtypeStruct((_BATCH, _WIDTH), jnp.float32),
        scratch_types=[
            pltpu.VMEM((chunk,), jnp.int32),
            pltpu.VMEM((chunk, _WIDTH), jnp.float32),
            pltpu.SemaphoreType.DMA,
        ],
    )
    def gather_kernel(idx_hbm, table_hbm, out_hbm, idx_v, rows_v, sem):
        wid = lax.axis_index("s") * nc + lax.axis_index("c")

        for ch in range(nch):
            base = wid * bpw + ch * chunk
            pltpu.sync_copy(idx_hbm.at[pl.ds(base, chunk)], idx_v)

            def fire(g, carry):
                v = idx_v[pl.ds(g * nl, nl)]
                for j in range(nl):
                    pltpu.async_copy(
                        table_hbm.at[v[j]], rows_v.at[g * nl + j], sem)
                return carry

            lax.fori_loop(0, chunk // nl, fire, 0)
            pltpu.make_async_copy(
                table_hbm.at[pl.ds(0, chunk)], rows_v, sem).wait()

            pltpu.sync_copy(rows_v, out_hbm.at[pl.ds(base, chunk)])

    return gather_kernel


_gather = _build_gather()


def kernel(y, genre_emb):
    out = _gather(y.astype(jnp.int32), genre_emb)
    return out[:, None, :]


# speedup vs baseline: 1.1804x; 1.1804x over previous
import functools

import jax
import jax.numpy as jnp
from jax import lax
from jax.experimental import pallas as pl
from jax.experimental.pallas import tpu as pltpu
from jax.experimental.pallas import tpu_sc as plsc

_BATCH = 16384
_WIDTH = 64
_LANES = 128


def _build_gather():
    info = plsc.get_sparse_core_info()
    nc, ns, nl = info.num_cores, info.num_subcores, info.num_lanes
    nw = nc * ns
    bpw = _BATCH // nw
    nb = 4
    ngrp = bpw // nl

    mesh = plsc.VectorSubcoreMesh(core_axis_name="c", subcore_axis_name="s")

    @functools.partial(
        pl.kernel,
        mesh=mesh,
        out_type=jax.ShapeDtypeStruct((_BATCH, _WIDTH), jnp.float32),
        scratch_types=[
            pltpu.VMEM((bpw,), jnp.int32),
            [pltpu.VMEM((_WIDTH, _LANES), jnp.float32)] * nb,
            pltpu.VMEM((bpw, _WIDTH), jnp.float32),
            pltpu.SemaphoreType.DMA,
        ],
        compiler_params=pltpu.CompilerParams(needs_layout_passes=False),
    )
    def gather_kernel(idx_hbm, tableT_hbm, out_hbm, idx_v, slabs, rows_v, sem):
        wid = lax.axis_index("s") * nc + lax.axis_index("c")
        base = wid * bpw
        pltpu.sync_copy(idx_hbm.at[pl.ds(base, bpw)], idx_v)

        cvecs = [lax.iota(jnp.int32, nl) + cc * nl
                 for cc in range(_WIDTH // nl)]

        def group(g, carry):
            v = idx_v[pl.ds(g * nl, nl)]
            for half in range(nl // nb):
                for b in range(nb):
                    y = v[half * nb + b]
                    t = lax.shift_right_logical(y, 7) * _LANES
                    pltpu.async_copy(
                        tableT_hbm.at[:, pl.ds(t, _LANES)], slabs[b], sem)
                for b in range(nb):
                    pltpu.make_async_copy(
                        tableT_hbm.at[:, pl.ds(0, _LANES)], slabs[b],
                        sem).wait()
                for b in range(nb):
                    y = v[half * nb + b]
                    lane = jnp.full((nl,), y & (_LANES - 1), jnp.int32)
                    kvec = jnp.full((nl,), g * nl + half * nb + b, jnp.int32)
                    for cc in range(_WIDTH // nl):
                        vals = plsc.load_gather(slabs[b], [cvecs[cc], lane])
                        plsc.store_scatter(
                            rows_v, [kvec, cvecs[cc]], vals)
            return carry

        lax.fori_loop(0, ngrp, group, 0)

        pltpu.sync_copy(rows_v, out_hbm.at[pl.ds(base, bpw)])

    return gather_kernel


_gather = _build_gather()


def kernel(y, genre_emb):
    out = _gather(y.astype(jnp.int32), genre_emb.T)
    return out[:, None, :]


# speedup vs baseline: 1.4802x; 1.2540x over previous
import functools

import jax
import jax.numpy as jnp
from jax import lax
from jax.experimental import pallas as pl
from jax.experimental.pallas import tpu as pltpu
from jax.experimental.pallas import tpu_sc as plsc

_BATCH = 16384
_WIDTH = 64
_LANES = 128


def _build_gather():
    info = plsc.get_sparse_core_info()
    nc, ns, nl = info.num_cores, info.num_subcores, info.num_lanes
    nw = nc * ns
    bpw = _BATCH // nw
    nb = 4
    nrounds = bpw // nb

    mesh = plsc.VectorSubcoreMesh(core_axis_name="c", subcore_axis_name="s")

    @functools.partial(
        pl.kernel,
        mesh=mesh,
        out_type=jax.ShapeDtypeStruct((_BATCH, _WIDTH), jnp.float32),
        scratch_types=[
            pltpu.VMEM((bpw,), jnp.int32),
            [pltpu.VMEM((_WIDTH, _LANES), jnp.float32)] * nb,
            pltpu.VMEM((bpw, _WIDTH), jnp.float32),
            [pltpu.SemaphoreType.DMA] * nb,
        ],
        compiler_params=pltpu.CompilerParams(needs_layout_passes=False),
    )
    def gather_kernel(idx_hbm, tableT_hbm, out_hbm, idx_v, slabs, rows_v, sems):
        wid = lax.axis_index("s") * nc + lax.axis_index("c")
        base = wid * bpw
        pltpu.sync_copy(idx_hbm.at[pl.ds(base, bpw)], idx_v)

        cvecs = [lax.iota(jnp.int32, nl) + cc * nl
                 for cc in range(_WIDTH // nl)]

        def fetch(k, b):
            yv = plsc.load_gather(idx_v, [jnp.full((nl,), k, jnp.int32)])
            t = lax.shift_right_logical(yv[0], 7) * _LANES
            pltpu.async_copy(
                tableT_hbm.at[:, pl.ds(t, _LANES)], slabs[b], sems[b])

        for b in range(nb):
            fetch(b, b)

        def round_(r, carry):
            for b in range(nb):
                k = r * nb + b
                pltpu.make_async_copy(
                    tableT_hbm.at[:, pl.ds(0, _LANES)], slabs[b],
                    sems[b]).wait()
                yv = plsc.load_gather(idx_v, [jnp.full((nl,), k, jnp.int32)])
                lane = jnp.full((nl,), yv[0] & (_LANES - 1), jnp.int32)
                kvec = jnp.full((nl,), k, jnp.int32)
                for cc in range(_WIDTH // nl):
                    vals = plsc.load_gather(slabs[b], [cvecs[cc], lane])
                    plsc.store_scatter(rows_v, [kvec, cvecs[cc]], vals)

                @pl.when(r < nrounds - 1)
                def _():
                    fetch(k + nb, b)

            return carry

        lax.fori_loop(0, nrounds, round_, 0)

        pltpu.sync_copy(rows_v, out_hbm.at[pl.ds(base, bpw)])

    return gather_kernel


_gather = _build_gather()


def kernel(y, genre_emb):
    out = _gather(y.astype(jnp.int32), genre_emb.T)
    return out[:, None, :]
